# baseline (speedup 1.0000x reference)
import functools

import jax
import jax.numpy as jnp
from jax import lax
from jax.experimental import pallas as pl
from jax.experimental.pallas import tpu as pltpu
from jax.experimental.pallas import tpu_sc as plsc

EMBED_DIM = 64
NUM_CORES = 2
NUM_SUBCORES = 16
NUM_WORKERS = NUM_CORES * NUM_SUBCORES
SEQ = 200
BL = 128
PER_W = BL * SEQ


def _emb_body(idx_hbm, table_hbm, out_hbm, idx_v, idxu_v, rows_v, po_v,
              sem_g, sem_o):
    wid = lax.axis_index("s") * NUM_CORES + lax.axis_index("c")
    base = wid * PER_W
    pltpu.sync_copy(idx_hbm.at[pl.ds(base, PER_W)], idx_v)

    lanes = lax.iota(jnp.int32, 16)
    lanes_seq = lanes * SEQ

    def unit(s, carry):
        for k in range(8):
            sel = lanes_seq + (16 * k * SEQ + s)
            vals = plsc.load_gather(idx_v, [sel])
            idxu_v[pl.ds(16 * k, 16)] = vals
        pltpu.async_copy(table_hbm.at[idxu_v], rows_v, sem_g).wait()
        for cb in range(8):
            for ci in range(8):
                c = jnp.full((16,), 8 * cb + ci, jnp.int32)
                for k in range(8):
                    rowsel = lanes + (16 * k)
                    v = plsc.load_gather(rows_v, [rowsel, c])
                    po_v[cb, 0, ci, pl.ds(16 * k, 16)] = v
        pltpu.async_copy(
            po_v, out_hbm.at[s, :, pl.ds(wid, 1)], sem_o).wait()
        return carry

    lax.fori_loop(0, SEQ, unit, 0)


def kernel(input_ids, weight):
    batch, seq = input_ids.shape
    n_flat = batch * seq
    idx_flat = input_ids.reshape(n_flat).astype(jnp.int32)

    mesh = plsc.VectorSubcoreMesh(core_axis_name="c", subcore_axis_name="s")
    emb = functools.partial(
        pl.kernel,
        mesh=mesh,
        out_type=jax.ShapeDtypeStruct(
            (SEQ, 8, NUM_WORKERS, 8, BL), jnp.float32),
        scratch_types=[
            pltpu.VMEM((PER_W,), jnp.int32),
            pltpu.VMEM((BL,), jnp.int32),
            pltpu.VMEM((BL, EMBED_DIM), jnp.float32),
            pltpu.VMEM((8, 1, 8, BL), jnp.float32),
            pltpu.SemaphoreType.DMA,
            pltpu.SemaphoreType.DMA,
        ],
        compiler_params=pltpu.CompilerParams(
            use_tc_tiling_on_sc=False, needs_layout_passes=False),
    )(_emb_body)

    out5 = emb(idx_flat, weight)
    return out5.transpose(2, 4, 0, 1, 3).reshape(batch, seq, EMBED_DIM)


# speedup vs baseline: 1.1426x; 1.1426x over previous
import functools

import jax
import jax.numpy as jnp
from jax import lax
from jax.experimental import pallas as pl
from jax.experimental.pallas import tpu as pltpu
from jax.experimental.pallas import tpu_sc as plsc

EMBED_DIM = 64
NUM_CORES = 2
NUM_SUBCORES = 16
NUM_WORKERS = NUM_CORES * NUM_SUBCORES
SEQ = 200
BL = 128
PER_W = BL * SEQ


def _emb_body(idx_hbm, table_hbm, out_hbm, idx_v, idxu_v, rows_v, po_v,
              sg0, sg1, so0, so1):
    wid = lax.axis_index("s") * NUM_CORES + lax.axis_index("c")
    base = wid * PER_W
    pltpu.sync_copy(idx_hbm.at[pl.ds(base, PER_W)], idx_v)

    sems_g = (sg0, sg1)
    sems_o = (so0, so1)
    lanes = lax.iota(jnp.int32, 16)
    lanes_seq = lanes * SEQ
    rowsels = [lanes + 16 * k for k in range(8)]
    csel = [jnp.full((16,), c, jnp.int32) for c in range(EMBED_DIM)]
    bsel = [jnp.full((16,), b, jnp.int32) for b in range(2)]

    def extract(s, b):
        for k in range(8):
            vals = plsc.load_gather(idx_v, [lanes_seq + (16 * k * SEQ + s)])
            idxu_v[b, pl.ds(16 * k, 16)] = vals

    def start_gather(b):
        pltpu.async_copy(table_hbm.at[idxu_v.at[b]], rows_v.at[b], sems_g[b])

    def wait_gather(b):
        pltpu.make_async_copy(
            table_hbm.at[idxu_v.at[b]], rows_v.at[b], sems_g[b]).wait()

    def transpose(bp):
        for cb in range(8):
            for ci in range(8):
                c = 8 * cb + ci
                for k in range(8):
                    v = plsc.load_gather(
                        rows_v, [bsel[bp], rowsels[k], csel[c]])
                    po_v[bp, cb, 0, ci, pl.ds(16 * k, 16)] = v

    def out_slot(sp):
        return out_hbm.at[sp, :, pl.ds(wid, 1)]

    def start_out(sp, bp):
        pltpu.async_copy(po_v.at[bp], out_slot(sp), sems_o[bp])

    def drain_out(bp):
        pltpu.make_async_copy(po_v.at[bp], out_slot(0), sems_o[bp]).wait()

    def retire(sp, bp, drain):
        wait_gather(bp)

        @pl.when(drain)
        def _():
            drain_out(bp)

        transpose(bp)
        start_out(sp, bp)

    def group(g, carry):
        for b in range(2):
            s = 2 * g + b
            extract(s, b)
            start_gather(b)
            if b == 0:
                @pl.when(g > 0)
                def _():
                    retire(s - 1, 1, g > 1)
            else:
                retire(s - 1, 0, g > 0)
        return carry

    lax.fori_loop(0, SEQ // 2, group, 0)

    retire(SEQ - 1, 1, jnp.bool_(True))
    drain_out(0)
    drain_out(1)


def kernel(input_ids, weight):
    batch, seq = input_ids.shape
    n_flat = batch * seq
    idx_flat = input_ids.reshape(n_flat).astype(jnp.int32)

    mesh = plsc.VectorSubcoreMesh(core_axis_name="c", subcore_axis_name="s")
    emb = functools.partial(
        pl.kernel,
        mesh=mesh,
        out_type=jax.ShapeDtypeStruct(
            (SEQ, 8, NUM_WORKERS, 8, BL), jnp.float32),
        scratch_types=[
            pltpu.VMEM((PER_W,), jnp.int32),
            pltpu.VMEM((2, BL), jnp.int32),
            pltpu.VMEM((2, BL, EMBED_DIM), jnp.float32),
            pltpu.VMEM((2, 8, 1, 8, BL), jnp.float32),
            pltpu.SemaphoreType.DMA,
            pltpu.SemaphoreType.DMA,
            pltpu.SemaphoreType.DMA,
            pltpu.SemaphoreType.DMA,
        ],
        compiler_params=pltpu.CompilerParams(
            use_tc_tiling_on_sc=False, needs_layout_passes=False),
    )(_emb_body)

    out5 = emb(idx_flat, weight)
    return out5.transpose(2, 4, 0, 1, 3).reshape(batch, seq, EMBED_DIM)


# speedup vs baseline: 1.3905x; 1.2170x over previous
import functools

import jax
import jax.numpy as jnp
from jax import lax
from jax.experimental import pallas as pl
from jax.experimental.pallas import tpu as pltpu
from jax.experimental.pallas import tpu_sc as plsc

EMBED_DIM = 64
NUM_CORES = 2
NUM_SUBCORES = 16
NUM_WORKERS = NUM_CORES * NUM_SUBCORES
SEQ = 200
BL = 128
PER_W = BL * SEQ


def _emb_body(idx_hbm, table_hbm, out_hbm, idx_v, idxu_v, rows_v, rows65_v,
              po_v, sg0, sg1, so0, so1):
    wid = lax.axis_index("s") * NUM_CORES + lax.axis_index("c")
    base = wid * PER_W
    pltpu.sync_copy(idx_hbm.at[pl.ds(base, PER_W)], idx_v)

    sems_g = (sg0, sg1)
    sems_o = (so0, so1)
    lanes = lax.iota(jnp.int32, 16)
    lanes_seq = lanes * SEQ
    rowsels = [lanes + 16 * k for k in range(8)]
    csel = [jnp.full((16,), c, jnp.int32) for c in range(EMBED_DIM)]
    bsel = [jnp.full((16,), b, jnp.int32) for b in range(2)]

    def extract(s, b):
        for k in range(8):
            vals = plsc.load_gather(idx_v, [lanes_seq + (16 * k * SEQ + s)])
            idxu_v[b, pl.ds(16 * k, 16)] = vals

    def start_gather(b):
        pltpu.async_copy(table_hbm.at[idxu_v.at[b]], rows_v.at[b], sems_g[b])

    def wait_gather(b):
        pltpu.make_async_copy(
            table_hbm.at[idxu_v.at[b]], rows_v.at[b], sems_g[b]).wait()

    def transpose(bp):
        for r in range(BL):
            for q in range(4):
                rows65_v[r, pl.ds(16 * q, 16)] = rows_v[bp, r, pl.ds(16 * q, 16)]
        for cb in range(8):
            for ci in range(8):
                c = 8 * cb + ci
                for k in range(8):
                    v = plsc.load_gather(rows65_v, [rowsels[k], csel[c]])
                    po_v[bp, cb, 0, ci, pl.ds(16 * k, 16)] = v

    def out_slot(sp):
        return out_hbm.at[sp, :, pl.ds(wid, 1)]

    def start_out(sp, bp):
        pltpu.async_copy(po_v.at[bp], out_slot(sp), sems_o[bp])

    def drain_out(bp):
        pltpu.make_async_copy(po_v.at[bp], out_slot(0), sems_o[bp]).wait()

    def retire(sp, bp, drain):
        wait_gather(bp)

        @pl.when(drain)
        def _():
            drain_out(bp)

        transpose(bp)
        start_out(sp, bp)

    def group(g, carry):
        for b in range(2):
            s = 2 * g + b
            extract(s, b)
            start_gather(b)
            if b == 0:
                @pl.when(g > 0)
                def _():
                    retire(s - 1, 1, g > 1)
            else:
                retire(s - 1, 0, g > 0)
        return carry

    lax.fori_loop(0, SEQ // 2, group, 0)

    retire(SEQ - 1, 1, jnp.bool_(True))
    drain_out(0)
    drain_out(1)


def kernel(input_ids, weight):
    batch, seq = input_ids.shape
    n_flat = batch * seq
    idx_flat = input_ids.reshape(n_flat).astype(jnp.int32)

    mesh = plsc.VectorSubcoreMesh(core_axis_name="c", subcore_axis_name="s")
    emb = functools.partial(
        pl.kernel,
        mesh=mesh,
        out_type=jax.ShapeDtypeStruct(
            (SEQ, 8, NUM_WORKERS, 8, BL), jnp.float32),
        scratch_types=[
            pltpu.VMEM((PER_W,), jnp.int32),
            pltpu.VMEM((2, BL), jnp.int32),
            pltpu.VMEM((2, BL, EMBED_DIM), jnp.float32),
            pltpu.VMEM((BL, EMBED_DIM + 1), jnp.float32),
            pltpu.VMEM((2, 8, 1, 8, BL), jnp.float32),
            pltpu.SemaphoreType.DMA,
            pltpu.SemaphoreType.DMA,
            pltpu.SemaphoreType.DMA,
            pltpu.SemaphoreType.DMA,
        ],
        compiler_params=pltpu.CompilerParams(
            use_tc_tiling_on_sc=False, needs_layout_passes=False),
    )(_emb_body)

    out5 = emb(idx_flat, weight)
    return out5.transpose(2, 4, 0, 1, 3).reshape(batch, seq, EMBED_DIM)


# speedup vs baseline: 1.8042x; 1.2975x over previous
import functools

import jax
import jax.numpy as jnp
from jax import lax
from jax.experimental import pallas as pl
from jax.experimental.pallas import tpu as pltpu
from jax.experimental.pallas import tpu_sc as plsc

EMBED_DIM = 64
NUM_CORES = 2
NUM_SUBCORES = 16
NUM_WORKERS = NUM_CORES * NUM_SUBCORES
CHUNK = 320
NBUF = 4
LAG = 2


def _emb_body(idx_hbm, table_hbm, out_hbm, idx_v, rows_v,
              sem_g0, sem_g1, sem_g2, sem_g3,
              sem_o0, sem_o1, sem_o2, sem_o3,
              *, per_w, n_chunk):
    wid = lax.axis_index("s") * NUM_CORES + lax.axis_index("c")
    base = wid * per_w
    pltpu.sync_copy(idx_hbm.at[pl.ds(base, per_w)], idx_v)

    sems_g = (sem_g0, sem_g1, sem_g2, sem_g3)
    sems_o = (sem_o0, sem_o1, sem_o2, sem_o3)
    n_groups = n_chunk // NBUF

    def out_slot(j):
        return out_hbm.at[pl.ds(base + j * CHUNK, CHUNK)]

    def drain_out(b):
        pltpu.make_async_copy(out_slot(0), rows_v.at[b], sems_o[b]).wait()

    def start_gather(j, b):
        pltpu.async_copy(
            table_hbm.at[idx_v.at[pl.ds(j * CHUNK, CHUNK)]],
            rows_v.at[b], sems_g[b])

    def wait_gather(b):
        pltpu.make_async_copy(
            table_hbm.at[idx_v.at[pl.ds(0, CHUNK)]],
            rows_v.at[b], sems_g[b]).wait()

    def retire(c, bc):
        wait_gather(bc)
        pltpu.async_copy(rows_v.at[bc], out_slot(c), sems_o[bc])

    def group(g, carry):
        for b in range(NBUF):
            j = g * NBUF + b

            @pl.when(g > 0)
            def _():
                drain_out(b)

            start_gather(j, b)

            bc = (b - LAG) % NBUF
            if b < LAG:
                @pl.when(g > 0)
                def _():
                    retire(j - LAG, bc)
            else:
                retire(j - LAG, bc)
        return carry

    lax.fori_loop(0, n_groups, group, 0)

    for c in range(n_chunk - LAG, n_chunk):
        retire(c, c % NBUF)
    for b in range(NBUF):
        drain_out(b)


def kernel(input_ids, weight):
    batch, seq = input_ids.shape
    n_flat = batch * seq
    per_w = n_flat // NUM_WORKERS
    n_chunk = per_w // CHUNK
    idx_flat = (input_ids.reshape(n_flat) * 2).astype(jnp.int32)
    vocab, _ = weight.shape
    w2 = jnp.pad(weight, ((0, 0), (0, EMBED_DIM))).reshape(
        2 * vocab, EMBED_DIM)

    mesh = plsc.VectorSubcoreMesh(core_axis_name="c", subcore_axis_name="s")
    emb = functools.partial(
        pl.kernel,
        mesh=mesh,
        out_type=jax.ShapeDtypeStruct((n_flat, EMBED_DIM), jnp.float32),
        scratch_types=[
            pltpu.VMEM((per_w,), jnp.int32),
            pltpu.VMEM((NBUF, CHUNK, EMBED_DIM), jnp.float32),
        ] + [pltpu.SemaphoreType.DMA] * (2 * NBUF),
        compiler_params=pltpu.CompilerParams(use_tc_tiling_on_sc=False),
    )(functools.partial(_emb_body, per_w=per_w, n_chunk=n_chunk))

    out = emb(idx_flat, w2)
    return out.reshape(batch, seq, EMBED_DIM)


# speedup vs baseline: 1.8068x; 1.0015x over previous
import functools

import jax
import jax.numpy as jnp
from jax import lax
from jax.experimental import pallas as pl
from jax.experimental.pallas import tpu as pltpu
from jax.experimental.pallas import tpu_sc as plsc

EMBED_DIM = 64
NUM_CORES = 2
NUM_SUBCORES = 16
NUM_WORKERS = NUM_CORES * NUM_SUBCORES
CHUNK = 320
NBUF = 4
LAG = 2


def _emb_body(idx_hbm, table_hbm, out_hbm, idx_v, rows_v,
              sem_g0, sem_g1, sem_g2, sem_g3,
              sem_o0, sem_o1, sem_o2, sem_o3,
              *, per_w, n_chunk):
    wid = lax.axis_index("s") * NUM_CORES + lax.axis_index("c")
    base = wid * per_w
    pltpu.sync_copy(idx_hbm.at[pl.ds(base, per_w)], idx_v)

    sems_g = (sem_g0, sem_g1, sem_g2, sem_g3)
    sems_o = (sem_o0, sem_o1, sem_o2, sem_o3)
    n_groups = n_chunk // NBUF

    def out_slot(j):
        return out_hbm.at[pl.ds(base + j * CHUNK, CHUNK)]

    def drain_out(b):
        pltpu.make_async_copy(out_slot(0), rows_v.at[b], sems_o[b]).wait()

    def start_gather(j, b):
        pltpu.async_copy(
            table_hbm.at[idx_v.at[pl.ds(j * CHUNK, CHUNK)]],
            rows_v.at[b], sems_g[b])

    def wait_gather(b):
        pltpu.make_async_copy(
            table_hbm.at[idx_v.at[pl.ds(0, CHUNK)]],
            rows_v.at[b], sems_g[b]).wait()

    def retire(c, bc):
        wait_gather(bc)
        pltpu.async_copy(rows_v.at[bc], out_slot(c), sems_o[bc])

    def group(g, carry):
        for b in range(NBUF):
            j = g * NBUF + b

            @pl.when(g > 0)
            def _():
                drain_out(b)

            start_gather(j, b)

            bc = (b - LAG) % NBUF
            if b < LAG:
                @pl.when(g > 0)
                def _():
                    retire(j - LAG, bc)
            else:
                retire(j - LAG, bc)
        return carry

    lax.fori_loop(0, n_groups, group, 0)

    for c in range(n_chunk - LAG, n_chunk):
        retire(c, c % NBUF)
    for b in range(NBUF):
        drain_out(b)


def kernel(input_ids, weight):
    batch, seq = input_ids.shape
    n_flat = batch * seq
    per_w = n_flat // NUM_WORKERS
    n_chunk = per_w // CHUNK
    idx_flat = (input_ids.reshape(n_flat) * 2).astype(jnp.int32)
    vocab, _ = weight.shape
    w2 = jnp.pad(weight, ((0, 0), (0, EMBED_DIM))).reshape(
        2 * vocab, EMBED_DIM)

    mesh = plsc.VectorSubcoreMesh(core_axis_name="c", subcore_axis_name="s")
    emb = functools.partial(
        pl.kernel,
        mesh=mesh,
        out_type=jax.ShapeDtypeStruct((n_flat, EMBED_DIM), jnp.float32),
        scratch_types=[
            pltpu.VMEM((per_w,), jnp.int32),
            pltpu.VMEM((NBUF, CHUNK, EMBED_DIM), jnp.float32),
        ] + [pltpu.SemaphoreType.DMA] * (2 * NBUF),
        compiler_params=pltpu.CompilerParams(use_tc_tiling_on_sc=False),
    )(functools.partial(_emb_body, per_w=per_w, n_chunk=n_chunk))

    out = emb(idx_flat, w2)
    out_b = jax.lax.optimization_barrier(out.reshape(n_flat // 2, 128))
    return out_b.reshape(batch, seq, EMBED_DIM)
